# Initial kernel scaffold; baseline (speedup 1.0000x reference)
import jax
import jax.numpy as jnp
from jax.experimental import pallas as pl


def kernel(hidden_states, W_g, b_g, W1, b1, W2, b2):
    raise NotImplementedError("write your pallas kernel here")


# speedup vs baseline: 1.7735x; 1.7735x over previous
import functools

import jax
import jax.numpy as jnp
from jax import lax
from jax.experimental import pallas as pl
from jax.experimental.pallas import tpu as pltpu
from jax.experimental.pallas import tpu_sc as plsc

E = 8
K = 2
D = 768
F = 3072
NT = 4096
NA = NT * K

TB = 512
NB = NT // TB

BM = 256
NBLK = NA // BM
T_TILES = NBLK + E - 1

NW = 32
TPW = NT // NW
SUB = 64


def _gate_body(x_ref, wg_ref, bg_ref,
               w0_ref, w1_ref, i0_ref, i1_ref, r0_ref, r1_ref, cnt_ref):
    x = x_ref[0]
    logits = jnp.dot(x, wg_ref[...], preferred_element_type=jnp.float32)
    logits = logits + bg_ref[...]
    m = jnp.max(logits, axis=-1, keepdims=True)
    z = jnp.exp(logits - m)
    p = z / jnp.sum(z, axis=-1, keepdims=True)

    idx8 = lax.broadcasted_iota(jnp.int32, (TB, E), 1)
    p0 = jnp.max(p, axis=-1, keepdims=True)
    i0 = jnp.min(jnp.where(p == p0, idx8, E), axis=-1, keepdims=True)
    oh0 = idx8 == i0
    pm = jnp.where(oh0, -jnp.inf, p)
    p1 = jnp.max(pm, axis=-1, keepdims=True)
    i1 = jnp.min(jnp.where(pm == p1, idx8, E), axis=-1, keepdims=True)
    oh1 = idx8 == i1

    s = p0 + p1
    ones16 = jnp.ones((1, 128), jnp.float32)
    w0_ref[...] = (p0 / s) * ones16
    w1_ref[...] = (p1 / s) * ones16
    i0_ref[...] = i0
    i1_ref[...] = i1

    sel = oh0.astype(jnp.float32) + oh1.astype(jnp.float32)
    r = lax.broadcasted_iota(jnp.int32, (TB, TB), 0)
    c = lax.broadcasted_iota(jnp.int32, (TB, TB), 1)
    tri = (r > c).astype(jnp.float32)
    pref = jnp.dot(tri, sel, preferred_element_type=jnp.float32)
    r0_ref[...] = jnp.sum(jnp.where(oh0, pref, 0.0), axis=-1,
                          keepdims=True).astype(jnp.int32)
    r1_ref[...] = jnp.sum(jnp.where(oh1, pref, 0.0), axis=-1,
                          keepdims=True).astype(jnp.int32)
    cnt_ref[...] = jnp.sum(sel, axis=0, keepdims=True)[None].astype(jnp.int32)


def _gating(x, w_g, b_g):
    wide_f = jax.ShapeDtypeStruct((NT, 128), jnp.float32)
    col_i = jax.ShapeDtypeStruct((NT, 1), jnp.int32)
    return pl.pallas_call(
        _gate_body,
        grid=(NB,),
        in_specs=[
            pl.BlockSpec((1, TB, D), lambda b: (b, 0, 0)),
            pl.BlockSpec((D, E), lambda b: (0, 0)),
            pl.BlockSpec((1, E), lambda b: (0, 0)),
        ],
        out_specs=[
            pl.BlockSpec((TB, 128), lambda b: (b, 0)),
            pl.BlockSpec((TB, 128), lambda b: (b, 0)),
            pl.BlockSpec((TB, 1), lambda b: (b, 0)),
            pl.BlockSpec((TB, 1), lambda b: (b, 0)),
            pl.BlockSpec((TB, 1), lambda b: (b, 0)),
            pl.BlockSpec((TB, 1), lambda b: (b, 0)),
            pl.BlockSpec((1, 1, E), lambda b: (b, 0, 0)),
        ],
        out_shape=[wide_f, wide_f, col_i, col_i, col_i, col_i,
                   jax.ShapeDtypeStruct((NB, 1, E), jnp.int32)],
    )(x.reshape(NB, TB, D), w_g, b_g.reshape(1, E))


def _metadata(cnt):
    cnt = cnt.reshape(NB, E)
    ex_cnt = jnp.sum(cnt, axis=0)
    off = jnp.concatenate([jnp.zeros((1,), jnp.int32),
                           jnp.cumsum(ex_cnt)]).astype(jnp.int32)
    carry = jnp.cumsum(cnt, axis=0) - cnt
    comb = (off[:E][None, :] + carry).reshape(NB, 1, E).astype(jnp.int32)

    fb = off[:E] // BM
    lb = (off[1:] + BM - 1) // BM
    nb = jnp.where(ex_cnt > 0, lb - fb, 0)
    cum = jnp.concatenate([jnp.zeros((1,), jnp.int32),
                           jnp.cumsum(nb)]).astype(jnp.int32)
    total = cum[E]
    s = jnp.arange(T_TILES, dtype=jnp.int32)
    eo = jnp.sum((s[:, None] >= cum[None, 1:]).astype(jnp.int32), axis=1)
    e_pad = jnp.max(jnp.where(nb > 0, jnp.arange(E, dtype=jnp.int32), 0))
    valid = s < total
    eo = jnp.where(valid, jnp.minimum(eo, E - 1), e_pad)
    bo = jnp.where(valid, fb[eo] + (s - cum[eo]), NBLK - 1)
    start = jnp.where(valid, jnp.maximum(off[eo], bo * BM), 0)
    end = jnp.where(valid, jnp.minimum(off[eo + 1], (bo + 1) * BM), 0)
    meta = jnp.stack([bo, eo, start, end]).astype(jnp.int32)
    return comb, meta


def _pos_body(i0_ref, i1_ref, r0_ref, r1_ref, comb_ref, p0_ref, p1_ref):
    comb = comb_ref[0]
    idx8 = lax.broadcasted_iota(jnp.int32, (TB, E), 1)
    oh0 = idx8 == i0_ref[...]
    oh1 = idx8 == i1_ref[...]
    base0 = jnp.sum(jnp.where(oh0, comb, 0), axis=-1, keepdims=True)
    base1 = jnp.sum(jnp.where(oh1, comb, 0), axis=-1, keepdims=True)
    p0_ref[...] = base0 + r0_ref[...]
    p1_ref[...] = base1 + r1_ref[...]


def _positions(i0, i1, r0, r1, comb):
    col_i = jax.ShapeDtypeStruct((NT, 1), jnp.int32)
    col_spec = pl.BlockSpec((TB, 1), lambda b: (b, 0))
    return pl.pallas_call(
        _pos_body,
        grid=(NB,),
        in_specs=[col_spec, col_spec, col_spec, col_spec,
                  pl.BlockSpec((1, 1, E), lambda b: (b, 0, 0))],
        out_specs=[col_spec, col_spec],
        out_shape=[col_i, col_i],
    )(i0, i1, r0, r1, comb)


@functools.cache
def _sc_dispatch_call():
    mesh = plsc.VectorSubcoreMesh(core_axis_name="c", subcore_axis_name="s")

    @functools.partial(
        pl.kernel,
        out_type=(jax.ShapeDtypeStruct((NA, D), jnp.float32),
                  jax.ShapeDtypeStruct((NA, 128), jnp.float32)),
        mesh=mesh,
        scratch_types=[
            pltpu.VMEM((SUB,), jnp.int32),
            pltpu.VMEM((SUB,), jnp.int32),
            pltpu.VMEM((SUB, D), jnp.float32),
            pltpu.VMEM((SUB, 128), jnp.float32),
            pltpu.VMEM((SUB, 128), jnp.float32),
            pltpu.SemaphoreType.DMA,
        ],
    )
    def dispatch(x_hbm, p0_hbm, p1_hbm, w0_hbm, w1_hbm, xs_hbm, ws_hbm,
                 p0_v, p1_v, rows_v, w0_v, w1_v, sem):
        wid = lax.axis_index("s") * 2 + lax.axis_index("c")
        for j in range(TPW // SUB):
            pltpu.sync_copy(p0_hbm.at[wid, j], p0_v)
            pltpu.sync_copy(p1_hbm.at[wid, j], p1_v)
            pltpu.sync_copy(w0_hbm.at[wid, j], w0_v)
            pltpu.sync_copy(w1_hbm.at[wid, j], w1_v)
            pltpu.sync_copy(x_hbm.at[pl.ds(wid * TPW + j * SUB, SUB)], rows_v)
            pltpu.async_copy(rows_v, xs_hbm.at[p0_v], sem).wait()
            pltpu.async_copy(rows_v, xs_hbm.at[p1_v], sem).wait()
            pltpu.async_copy(w0_v, ws_hbm.at[p0_v], sem).wait()
            pltpu.async_copy(w1_v, ws_hbm.at[p1_v], sem).wait()

    return dispatch


def _sc_dispatch(x, p0w, p1w, w0w, w1w):
    nj = TPW // SUB
    return _sc_dispatch_call()(x, p0w.reshape(NW, nj, SUB),
                               p1w.reshape(NW, nj, SUB),
                               w0w.reshape(NW, nj, SUB, 128),
                               w1w.reshape(NW, nj, SUB, 128))


def _gmm_body(meta_ref, x_ref, ws_ref, w1_ref, b1_ref, w2_ref, b2_ref, o_ref):
    t = pl.program_id(0)
    bid = meta_ref[0, t]
    start = meta_ref[2, t]
    end = meta_ref[3, t]
    h = jnp.dot(x_ref[...], w1_ref[0], preferred_element_type=jnp.float32)
    h = jax.nn.gelu(h + b1_ref[0])
    acc = jnp.dot(h, w2_ref[0], preferred_element_type=jnp.float32)
    acc = (acc + b2_ref[0]) * ws_ref[...][:, :1]
    rid = bid * BM + lax.broadcasted_iota(jnp.int32, (BM, 1), 0)
    mask = (rid >= start) & (rid < end)
    o_ref[...] = jnp.where(mask, acc, o_ref[...])


def _gmm(meta, xs, ws, w1, b1, w2, b2):
    grid_spec = pltpu.PrefetchScalarGridSpec(
        num_scalar_prefetch=1,
        grid=(T_TILES,),
        in_specs=[
            pl.BlockSpec((BM, D), lambda t, m: (m[0, t], 0)),
            pl.BlockSpec((BM, 128), lambda t, m: (m[0, t], 0)),
            pl.BlockSpec((1, D, F), lambda t, m: (m[1, t], 0, 0)),
            pl.BlockSpec((1, 1, F), lambda t, m: (m[1, t], 0, 0)),
            pl.BlockSpec((1, F, D), lambda t, m: (m[1, t], 0, 0)),
            pl.BlockSpec((1, 1, D), lambda t, m: (m[1, t], 0, 0)),
        ],
        out_specs=pl.BlockSpec((BM, D), lambda t, m: (m[0, t], 0)),
    )
    return pl.pallas_call(
        _gmm_body,
        grid_spec=grid_spec,
        out_shape=jax.ShapeDtypeStruct((NA, D), jnp.float32),
        compiler_params=pltpu.CompilerParams(
            dimension_semantics=("arbitrary",)),
    )(meta, xs, ws, w1, b1.reshape(E, 1, F), w2, b2.reshape(E, 1, D))


@functools.cache
def _sc_combine_call():
    mesh = plsc.VectorSubcoreMesh(core_axis_name="c", subcore_axis_name="s")

    @functools.partial(
        pl.kernel,
        out_type=jax.ShapeDtypeStruct((NT, D), jnp.float32),
        mesh=mesh,
        scratch_types=[
            pltpu.VMEM((TPW,), jnp.int32),
            pltpu.VMEM((TPW,), jnp.int32),
            pltpu.VMEM((SUB, D), jnp.float32),
            pltpu.VMEM((SUB, D), jnp.float32),
            pltpu.SemaphoreType.DMA,
        ],
    )
    def combine(y_hbm, p0_hbm, p1_hbm, o_hbm, p0_v, p1_v, buf0, buf1, sem):
        wid = lax.axis_index("s") * 2 + lax.axis_index("c")
        pltpu.sync_copy(p0_hbm.at[wid], p0_v)
        pltpu.sync_copy(p1_hbm.at[wid], p1_v)
        for j in range(TPW // SUB):
            pltpu.async_copy(y_hbm.at[p0_v.at[pl.ds(j * SUB, SUB)]], buf0,
                             sem).wait()
            pltpu.async_copy(y_hbm.at[p1_v.at[pl.ds(j * SUB, SUB)]], buf1,
                             sem).wait()

            def row_body(i, _):
                for cv in range(D // 16):
                    sl = pl.ds(cv * 16, 16)
                    buf0[i, sl] = buf0[i, sl] + buf1[i, sl]
                return 0

            lax.fori_loop(0, SUB, row_body, 0)
            pltpu.sync_copy(buf0, o_hbm.at[pl.ds(wid * TPW + j * SUB, SUB)])

    return combine


def _sc_combine(y, p0w, p1w):
    return _sc_combine_call()(y, p0w, p1w)


def kernel(hidden_states, W_g, b_g, W1, b1, W2, b2):
    B, S, _ = hidden_states.shape
    x = hidden_states.reshape(NT, D)

    w0, w1, i0, i1, r0, r1, cnt = _gating(x, W_g, b_g)
    comb, meta = _metadata(cnt)
    p0, p1 = _positions(i0, i1, r0, r1, comb)

    p0w = p0.reshape(NW, TPW)
    p1w = p1.reshape(NW, TPW)
    xs, ws = _sc_dispatch(x, p0w, p1w,
                          w0.reshape(NW, TPW, 128), w1.reshape(NW, TPW, 128))
    y = _gmm(meta, xs, ws, W1, b1, W2, b2)
    out = _sc_combine(y, p0w, p1w)
    return out.reshape(B, S, D)
